# Initial kernel scaffold; baseline (speedup 1.0000x reference)
import jax
import jax.numpy as jnp
from jax.experimental import pallas as pl


def kernel(layer_attention_probes, mask):
    raise NotImplementedError("write your pallas kernel here")


# speedup vs baseline: 1.2081x; 1.2081x over previous
import functools

import jax
import jax.numpy as jnp
from jax.experimental import pallas as pl
from jax.experimental.pallas import tpu as pltpu

ALPHA_C = 1.5
S = 2048
H = 12
QTOT = H * S
QBLK = 512
NQ = QTOT // QBLK


def _body(x_ref, m_ref, o_ref, acc_ref):
    i = pl.program_id(1)

    @pl.when(i == 0)
    def _init():
        acc_ref[...] = jnp.zeros_like(acc_ref)

    x = x_ref[0]
    part = jnp.zeros((8, S), jnp.float32)
    for k in range(QBLK // 8):
        part = part + x[k * 8:(k + 1) * 8, :]
    acc_ref[...] += part

    @pl.when(i == NQ - 1)
    def _epilogue():
        colsum = jnp.sum(acc_ref[...], axis=0, keepdims=True)
        scores = colsum * jnp.float32(1.0 / QTOT)
        m = m_ref[0]
        sep_i = (jnp.sum(m) - 1.0).astype(jnp.int32)
        idx = jax.lax.broadcasted_iota(jnp.int32, (1, S), 1)
        valid = (idx >= 1) & (idx <= sep_i - 1)
        n = (sep_i - 1).astype(jnp.float32)
        mean = jnp.sum(jnp.where(valid, scores, 0.0)) / n
        dev = jnp.where(valid, scores - mean, 0.0)
        var = jnp.sum(dev * dev) / (n - 1.0)
        std = jnp.sqrt(var)
        lo = mean - jnp.float32(ALPHA_C) * std
        hi = mean + jnp.float32(ALPHA_C) * std
        keep = valid & (scores >= lo) & (scores <= hi)
        out = jnp.where(keep | (idx == 0) | (idx == sep_i), 1.0, 0.0)
        o_ref[0] = out.astype(jnp.float32)


@functools.partial(jax.jit, static_argnames=())
def kernel(layer_attention_probes, mask):
    b = layer_attention_probes.shape[0]
    x3 = layer_attention_probes.reshape(b, QTOT, S)
    mask3 = mask.reshape(b, 1, S)
    out = pl.pallas_call(
        _body,
        grid=(b, NQ),
        in_specs=[
            pl.BlockSpec((1, QBLK, S), lambda bi, qi: (bi, qi, 0)),
            pl.BlockSpec((1, 1, S), lambda bi, qi: (bi, 0, 0)),
        ],
        out_specs=pl.BlockSpec((1, 1, S), lambda bi, qi: (bi, 0, 0)),
        out_shape=jax.ShapeDtypeStruct((b, 1, S), jnp.float32),
        scratch_shapes=[pltpu.VMEM((8, S), jnp.float32)],
        compiler_params=pltpu.CompilerParams(
            dimension_semantics=("arbitrary", "arbitrary"),
        ),
    )(x3, mask3)
    return out.reshape(b, S)


# speedup vs baseline: 1.2952x; 1.0721x over previous
import functools

import jax
import jax.numpy as jnp
from jax.experimental import pallas as pl
from jax.experimental.pallas import tpu as pltpu

ALPHA_C = 1.5
S = 2048
H = 12
QTOT = H * S
QBLK = 1024
NQ = QTOT // QBLK


def _body(x_ref, m_ref, o_ref, acc_ref):
    i = pl.program_id(1)

    @pl.when(i == 0)
    def _init():
        acc_ref[...] = jnp.zeros_like(acc_ref)

    x = x_ref[0]
    part = jnp.zeros((8, S), jnp.float32)
    for k in range(QBLK // 8):
        part = part + x[k * 8:(k + 1) * 8, :]
    acc_ref[...] += part

    @pl.when(i == NQ - 1)
    def _epilogue():
        colsum = jnp.sum(acc_ref[...], axis=0, keepdims=True)
        scores = colsum * jnp.float32(1.0 / QTOT)
        m = m_ref[0]
        sep_i = (jnp.sum(m) - 1.0).astype(jnp.int32)
        idx = jax.lax.broadcasted_iota(jnp.int32, (1, S), 1)
        valid = (idx >= 1) & (idx <= sep_i - 1)
        n = (sep_i - 1).astype(jnp.float32)
        mean = jnp.sum(jnp.where(valid, scores, 0.0)) / n
        dev = jnp.where(valid, scores - mean, 0.0)
        var = jnp.sum(dev * dev) / (n - 1.0)
        std = jnp.sqrt(var)
        lo = mean - jnp.float32(ALPHA_C) * std
        hi = mean + jnp.float32(ALPHA_C) * std
        keep = valid & (scores >= lo) & (scores <= hi)
        out = jnp.where(keep | (idx == 0) | (idx == sep_i), 1.0, 0.0)
        o_ref[0] = out.astype(jnp.float32)


@functools.partial(jax.jit, static_argnames=())
def kernel(layer_attention_probes, mask):
    b = layer_attention_probes.shape[0]
    x3 = layer_attention_probes.reshape(b, QTOT, S)
    mask3 = mask.reshape(b, 1, S)
    out = pl.pallas_call(
        _body,
        grid=(b, NQ),
        in_specs=[
            pl.BlockSpec((1, QBLK, S), lambda bi, qi: (bi, qi, 0)),
            pl.BlockSpec((1, 1, S), lambda bi, qi: (bi, 0, 0)),
        ],
        out_specs=pl.BlockSpec((1, 1, S), lambda bi, qi: (bi, 0, 0)),
        out_shape=jax.ShapeDtypeStruct((b, 1, S), jnp.float32),
        scratch_shapes=[pltpu.VMEM((8, S), jnp.float32)],
        compiler_params=pltpu.CompilerParams(
            dimension_semantics=("arbitrary", "arbitrary"),
        ),
    )(x3, mask3)
    return out.reshape(b, S)
